# Initial kernel scaffold; baseline (speedup 1.0000x reference)
import jax
import jax.numpy as jnp
from jax.experimental import pallas as pl


def kernel(z, W):
    raise NotImplementedError("write your pallas kernel here")


# speedup vs baseline: 1.2461x; 1.2461x over previous
import jax
import jax.numpy as jnp
from jax import lax
from jax.experimental import pallas as pl

_NE = 1024
_D = 64
_B = 16
_P = 1024
_NPIX = _B * _P
_NELEM = _B * _D * _P


def _vq_body(z_ref, w_ref, wt_ref, w2_ref, out_ref, idx_ref, cnt_ref,
             loss_ref, perp_ref, util_ref):
    b = pl.program_id(0)
    zr = z_ref[0]
    wv = w_ref[...]
    wt = wt_ref[...]

    z2 = jnp.sum(zr * zr, axis=0, keepdims=True)
    w2 = jnp.sum(wv * wv, axis=1, keepdims=True)
    m2 = lax.dot(w2_ref[...], zr,
                 preferred_element_type=jnp.float32)
    d = (z2 - m2) + w2

    dmin = jnp.min(d, axis=0, keepdims=True)
    iota0 = lax.broadcasted_iota(jnp.int32, (_NE, _P), 0)
    idx = jnp.min(jnp.where(d == dmin, iota0, _NE),
                  axis=0, keepdims=True)
    idx_ref[0] = idx

    e = jnp.where(iota0 == idx, 1.0, 0.0)
    q = lax.dot(wt, e, preferred_element_type=jnp.float32)
    out_ref[0] = zr + (q - zr)

    ls = jnp.sum((q - zr) ** 2, axis=(0, 1), keepdims=True)
    cnt = jnp.sum(e, axis=1, keepdims=True)

    @pl.when(b == 0)
    def _():
        cnt_ref[...] = cnt
        loss_ref[...] = ls

    @pl.when(b > 0)
    def _():
        cnt_ref[...] += cnt
        loss_ref[...] += ls

    @pl.when(b == _B - 1)
    def _():
        mean = loss_ref[...] * (1.0 / _NELEM)
        loss_ref[...] = mean + 0.25 * mean
        p = cnt_ref[...] * (1.0 / _NPIX)
        plog = p * jnp.log(p + 1e-10)
        perp_ref[...] = jnp.exp(-jnp.sum(plog, axis=(0, 1), keepdims=True))
        util_ref[...] = jnp.sum(jnp.where(p > 0, 1.0, 0.0),
                                axis=(0, 1), keepdims=True) * (1.0 / _NE)


def _vq_pallas(z3, w, wt, w2x):
    return pl.pallas_call(
        _vq_body,
        grid=(_B,),
        in_specs=[
            pl.BlockSpec((1, _D, _P), lambda b: (b, 0, 0)),
            pl.BlockSpec((_NE, _D), lambda b: (0, 0)),
            pl.BlockSpec((_D, _NE), lambda b: (0, 0)),
            pl.BlockSpec((_NE, _D), lambda b: (0, 0)),
        ],
        out_specs=[
            pl.BlockSpec((1, _D, _P), lambda b: (b, 0, 0)),
            pl.BlockSpec((1, 1, _P), lambda b: (b, 0, 0)),
            pl.BlockSpec((_NE, 1), lambda b: (0, 0)),
            pl.BlockSpec((1, 1), lambda b: (0, 0)),
            pl.BlockSpec((1, 1), lambda b: (0, 0)),
            pl.BlockSpec((1, 1), lambda b: (0, 0)),
        ],
        out_shape=[
            jax.ShapeDtypeStruct((_B, _D, _P), jnp.float32),
            jax.ShapeDtypeStruct((_B, 1, _P), jnp.int32),
            jax.ShapeDtypeStruct((_NE, 1), jnp.float32),
            jax.ShapeDtypeStruct((1, 1), jnp.float32),
            jax.ShapeDtypeStruct((1, 1), jnp.float32),
            jax.ShapeDtypeStruct((1, 1), jnp.float32),
        ],
    )(z3, w, wt, w2x)


def kernel(z, W):
    z3 = z.reshape(_B, _D, _P)
    qst3, idx3, _cnt, loss, perp, util = _vq_pallas(z3, W, W.T, W + W)
    quantized_st = qst3.reshape(z.shape)
    encoding_indices = idx3.reshape(_NPIX)
    return (quantized_st, loss[0, 0], perp[0, 0], util[0, 0],
            encoding_indices)
